# baseline (speedup 1.0000x reference)
import functools

import jax
import jax.numpy as jnp
from jax import lax
from jax.experimental import pallas as pl
from jax.experimental.pallas import tpu as pltpu
from jax.experimental.pallas import tpu_sc as plsc


def _make_sc_interp(total, C, H, W, N):
    info = plsc.get_sparse_core_info()
    NC, NS, L = info.num_cores, info.num_subcores, info.num_lanes
    NW = NC * NS
    assert total % NW == 0
    per_w = total // NW
    assert N % per_w == 0
    K = 32
    nchunk = per_w // K
    creg = C // L

    mesh = plsc.VectorSubcoreMesh(core_axis_name="c", subcore_axis_name="s")

    def body(table, px_hbm, py_hbm, out_hbm,
             px_v, py_v, idx_v, w00_v, w01_v, w10_v, w11_v, gbuf, obuf, sem):
        cid = lax.axis_index("c")
        sid = lax.axis_index("s")
        wid = sid * NC + cid
        base_n = wid * per_w
        row_base = (base_n // N) * (H * W)

        pltpu.sync_copy(px_hbm.at[pl.ds(base_n, per_w)], px_v)
        pltpu.sync_copy(py_hbm.at[pl.ds(base_n, per_w)], py_v)

        fW = float(W)
        fH = float(H)

        def idxw(g, carry):
            px = px_v[pl.ds(g * L, L)]
            py = py_v[pl.ds(g * L, L)]
            ix = jnp.minimum(jnp.maximum(px * fW - 0.5, 0.0), fW - 1.0)
            iy = jnp.minimum(jnp.maximum(py * fH - 0.5, 0.0), fH - 1.0)
            x0 = ix.astype(jnp.int32)
            y0 = iy.astype(jnp.int32)
            wx1 = ix - x0.astype(jnp.float32)
            wy1 = iy - y0.astype(jnp.float32)
            wx0 = 1.0 - wx1
            wy0 = 1.0 - wy1
            x1 = jnp.minimum(x0 + 1, W - 1)
            y1 = jnp.minimum(y0 + 1, H - 1)
            r0 = row_base + y0 * W
            r1 = row_base + y1 * W
            t = g // 2
            h = (g % 2) * L
            idx_v[t, pl.ds(0 * K + h, L)] = r0 + x0
            idx_v[t, pl.ds(1 * K + h, L)] = r0 + x1
            idx_v[t, pl.ds(2 * K + h, L)] = r1 + x0
            idx_v[t, pl.ds(3 * K + h, L)] = r1 + x1
            w00_v[pl.ds(g * L, L)] = wy0 * wx0
            w01_v[pl.ds(g * L, L)] = wy0 * wx1
            w10_v[pl.ds(g * L, L)] = wy1 * wx0
            w11_v[pl.ds(g * L, L)] = wy1 * wx1
            return carry

        lax.fori_loop(0, per_w // L, idxw, 0)

        def chunk(t, carry):
            pltpu.async_copy(table.at[idx_v.at[t]], gbuf, sem).wait()

            for h in range(K // L):
                wbase = t * K + h * L
                wv00 = w00_v[pl.ds(wbase, L)]
                wv01 = w01_v[pl.ds(wbase, L)]
                wv10 = w10_v[pl.ds(wbase, L)]
                wv11 = w11_v[pl.ds(wbase, L)]
                for i2 in range(L):
                    i = h * L + i2
                    a, b, c, d = wv00[i2], wv01[i2], wv10[i2], wv11[i2]
                    for r in range(creg):
                        s = pl.ds(r * L, L)
                        acc = (gbuf[i, s] * a + gbuf[K + i, s] * b
                               + gbuf[2 * K + i, s] * c + gbuf[3 * K + i, s] * d)
                        obuf[i, s] = acc

            pltpu.sync_copy(obuf, out_hbm.at[pl.ds(base_n + t * K, K)])
            return carry

        lax.fori_loop(0, nchunk, chunk, 0)

    return pl.kernel(
        body,
        out_type=jax.ShapeDtypeStruct((total, C), jnp.float32),
        mesh=mesh,
        compiler_params=pltpu.CompilerParams(use_tc_tiling_on_sc=False),
        scratch_types=[
            pltpu.VMEM((per_w,), jnp.float32),
            pltpu.VMEM((per_w,), jnp.float32),
            pltpu.VMEM((nchunk, 4 * K), jnp.int32),
            pltpu.VMEM((per_w,), jnp.float32),
            pltpu.VMEM((per_w,), jnp.float32),
            pltpu.VMEM((per_w,), jnp.float32),
            pltpu.VMEM((per_w,), jnp.float32),
            pltpu.VMEM((4 * K, C), jnp.float32),
            pltpu.VMEM((K, C), jnp.float32),
            pltpu.SemaphoreType.DMA,
        ],
    )


@jax.jit
def _run(features, positions):
    B, C, H, W = features.shape
    N = positions.shape[1]
    table = jnp.transpose(features, (0, 2, 3, 1)).reshape(B * H * W, C)
    px = positions[..., 0].reshape(-1)
    py = positions[..., 1].reshape(-1)
    out = _make_sc_interp(B * N, C, H, W, N)(table, px, py)
    return out.reshape(B, N, C)


def kernel(features, positions):
    return _run(features, positions)


# speedup vs baseline: 1.0159x; 1.0159x over previous
import functools

import jax
import jax.numpy as jnp
from jax import lax
from jax.experimental import pallas as pl
from jax.experimental.pallas import tpu as pltpu
from jax.experimental.pallas import tpu_sc as plsc


def _make_sc_interp(total, C, H, W, N):
    info = plsc.get_sparse_core_info()
    NC, NS, L = info.num_cores, info.num_subcores, info.num_lanes
    NW = NC * NS
    assert total % NW == 0
    per_w = total // NW
    assert N % per_w == 0
    K = 32
    nchunk = per_w // K
    creg = C // L

    mesh = plsc.VectorSubcoreMesh(core_axis_name="c", subcore_axis_name="s")

    def body(table, px_hbm, py_hbm, out_hbm,
             px_v, py_v, idx_v, w00_v, w01_v, w10_v, w11_v,
             gbuf0, gbuf1, obuf0, obuf1, gsem0, gsem1, osem0, osem1):
        cid = lax.axis_index("c")
        sid = lax.axis_index("s")
        wid = sid * NC + cid
        base_n = wid * per_w
        row_base = (base_n // N) * (H * W)

        pltpu.sync_copy(px_hbm.at[pl.ds(base_n, per_w)], px_v)
        pltpu.sync_copy(py_hbm.at[pl.ds(base_n, per_w)], py_v)

        fW = float(W)
        fH = float(H)

        def idxw(g, carry):
            px = px_v[pl.ds(g * L, L)]
            py = py_v[pl.ds(g * L, L)]
            ix = jnp.minimum(jnp.maximum(px * fW - 0.5, 0.0), fW - 1.0)
            iy = jnp.minimum(jnp.maximum(py * fH - 0.5, 0.0), fH - 1.0)
            x0 = ix.astype(jnp.int32)
            y0 = iy.astype(jnp.int32)
            wx1 = ix - x0.astype(jnp.float32)
            wy1 = iy - y0.astype(jnp.float32)
            wx0 = 1.0 - wx1
            wy0 = 1.0 - wy1
            x1 = jnp.minimum(x0 + 1, W - 1)
            y1 = jnp.minimum(y0 + 1, H - 1)
            r0 = row_base + y0 * W
            r1 = row_base + y1 * W
            t = g // 2
            h = (g % 2) * L
            idx_v[t, pl.ds(0 * K + h, L)] = r0 + x0
            idx_v[t, pl.ds(1 * K + h, L)] = r0 + x1
            idx_v[t, pl.ds(2 * K + h, L)] = r1 + x0
            idx_v[t, pl.ds(3 * K + h, L)] = r1 + x1
            w00_v[pl.ds(g * L, L)] = wy0 * wx0
            w01_v[pl.ds(g * L, L)] = wy0 * wx1
            w10_v[pl.ds(g * L, L)] = wy1 * wx0
            w11_v[pl.ds(g * L, L)] = wy1 * wx1
            return carry

        lax.fori_loop(0, per_w // L, idxw, 0)

        pltpu.async_copy(table.at[idx_v.at[0]], gbuf0, gsem0)

        def outer(to, carry):
            for par in range(2):
                t = to * 2 + par
                gbuf = gbuf0 if par == 0 else gbuf1
                gsem = gsem0 if par == 0 else gsem1
                gbuf_n = gbuf1 if par == 0 else gbuf0
                gsem_n = gsem1 if par == 0 else gsem0
                obuf = obuf0 if par == 0 else obuf1
                osem = osem0 if par == 0 else osem1

                @pl.when(t + 1 < nchunk)
                def _():
                    pltpu.async_copy(table.at[idx_v.at[t + 1]], gbuf_n, gsem_n)

                pltpu.make_async_copy(table.at[idx_v.at[t]], gbuf, gsem).wait()

                @pl.when(t >= 2)
                def _():
                    pltpu.make_async_copy(
                        obuf, out_hbm.at[pl.ds(base_n + (t - 2) * K, K)], osem
                    ).wait()

                for h in range(K // L):
                    wbase = t * K + h * L
                    wv00 = w00_v[pl.ds(wbase, L)]
                    wv01 = w01_v[pl.ds(wbase, L)]
                    wv10 = w10_v[pl.ds(wbase, L)]
                    wv11 = w11_v[pl.ds(wbase, L)]
                    for i2 in range(L):
                        i = h * L + i2
                        a, b, c, d = wv00[i2], wv01[i2], wv10[i2], wv11[i2]
                        for r in range(creg):
                            s = pl.ds(r * L, L)
                            acc = (gbuf[i, s] * a + gbuf[K + i, s] * b
                                   + gbuf[2 * K + i, s] * c + gbuf[3 * K + i, s] * d)
                            obuf[i, s] = acc

                pltpu.async_copy(obuf, out_hbm.at[pl.ds(base_n + t * K, K)], osem)
            return carry

        lax.fori_loop(0, nchunk // 2, outer, 0)
        pltpu.make_async_copy(
            obuf0, out_hbm.at[pl.ds(base_n + (nchunk - 2) * K, K)], osem0
        ).wait()
        pltpu.make_async_copy(
            obuf1, out_hbm.at[pl.ds(base_n + (nchunk - 1) * K, K)], osem1
        ).wait()

    return pl.kernel(
        body,
        out_type=jax.ShapeDtypeStruct((total, C), jnp.float32),
        mesh=mesh,
        compiler_params=pltpu.CompilerParams(use_tc_tiling_on_sc=False),
        scratch_types=[
            pltpu.VMEM((per_w,), jnp.float32),
            pltpu.VMEM((per_w,), jnp.float32),
            pltpu.VMEM((nchunk, 4 * K), jnp.int32),
            pltpu.VMEM((per_w,), jnp.float32),
            pltpu.VMEM((per_w,), jnp.float32),
            pltpu.VMEM((per_w,), jnp.float32),
            pltpu.VMEM((per_w,), jnp.float32),
            pltpu.VMEM((4 * K, C), jnp.float32),
            pltpu.VMEM((4 * K, C), jnp.float32),
            pltpu.VMEM((K, C), jnp.float32),
            pltpu.VMEM((K, C), jnp.float32),
            pltpu.SemaphoreType.DMA,
            pltpu.SemaphoreType.DMA,
            pltpu.SemaphoreType.DMA,
            pltpu.SemaphoreType.DMA,
        ],
    )


@jax.jit
def _run(features, positions):
    B, C, H, W = features.shape
    N = positions.shape[1]
    table = jnp.transpose(features, (0, 2, 3, 1)).reshape(B * H * W, C)
    px = positions[..., 0].reshape(-1)
    py = positions[..., 1].reshape(-1)
    out = _make_sc_interp(B * N, C, H, W, N)(table, px, py)
    return out.reshape(B, N, C)


def kernel(features, positions):
    return _run(features, positions)
